# baseline (speedup 1.0000x reference)
import functools

import jax
import jax.numpy as jnp
from jax.experimental import pallas as pl
from jax.experimental.pallas import tpu as pltpu

H = 256
G3 = 3 * H
NCLS = 80
F32 = jnp.float32
BF16 = jnp.bfloat16


def _gru_update(gi, mm, bhn, h):
    grz = gi[:, :2 * H] + mm[:, :2 * H]
    r = jax.nn.sigmoid(grz[:, :H])
    z = jax.nn.sigmoid(grz[:, H:])
    n = jnp.tanh(gi[:, 2 * H:] + r * (mm[:, 2 * H:] + bhn))
    return z * (h - n) + n


def _layer_body(n_in, Tc, K, *refs):
    k = 0
    fwd = refs[k:k + n_in]; k += n_in
    rev = refs[k:k + n_in]; k += n_in
    Wf = refs[k:k + n_in]; k += n_in
    Wr = refs[k:k + n_in]; k += n_in
    WhhTf, WhhTr, bihf, bihr, bhhf, bhhr = refs[k:k + 6]; k += 6
    out_f, out_r = refs[k:k + 2]; k += 2
    gif_sc, gir_sc, hf_sc, hr_sc = refs[k:k + 4]

    B = out_f.shape[1]

    @pl.when(pl.program_id(0) == 0)
    def _():
        hf_sc[...] = jnp.zeros_like(hf_sc)
        hr_sc[...] = jnp.zeros_like(hr_sc)

    bf = bihf[...]
    br = bihr[...]
    wfv = [w[...] for w in Wf]
    wrv = [w[...] for w in Wr]

    gif = bf
    for a, wv in zip(fwd, wfv):
        x2 = a[0:K].reshape(K * B, a.shape[2])
        gif = gif + jnp.dot(x2, wv, preferred_element_type=F32)
    gif_sc[0:K] = gif.reshape(K, B, G3)

    gir = br
    for a, wv in zip(rev, wrv):
        x2 = a[Tc - K:Tc].reshape(K * B, a.shape[2])
        gir = gir + jnp.dot(x2, wv, preferred_element_type=F32)
    gir_sc[Tc - K:Tc] = gir.reshape(K, B, G3)

    whf = WhhTf[...]
    whr = WhhTr[...]
    bhnf = bhhf[...]
    bhnr = bhhr[...]

    def step(s, carry):
        hf, hr = carry
        mmf = jnp.dot(hf.astype(BF16), whf, preferred_element_type=F32)
        mmr = jnp.dot(hr.astype(BF16), whr, preferred_element_type=F32)
        hf = _gru_update(gif_sc[s], mmf, bhnf, hf)
        hr = _gru_update(gir_sc[Tc - 1 - s], mmr, bhnr, hr)
        out_f[s] = hf.astype(BF16)
        out_r[Tc - 1 - s] = hr.astype(BF16)
        t2 = jnp.minimum(s + K, Tc - 1)
        acc = bf
        for a, wv in zip(fwd, wfv):
            acc = acc + jnp.dot(a[t2], wv, preferred_element_type=F32)
        gif_sc[t2] = acc
        t2r = Tc - 1 - t2
        accr = br
        for a, wv in zip(rev, wrv):
            accr = accr + jnp.dot(a[t2r], wv, preferred_element_type=F32)
        gir_sc[t2r] = accr
        return hf, hr

    hf, hr = jax.lax.fori_loop(0, Tc, step, (hf_sc[...], hr_sc[...]),
                               unroll=4)
    hf_sc[...] = hf
    hr_sc[...] = hr


def _bigru_layer(inputs, Wf_list, Wr_list, WhhTf, WhhTr, bihf, bihr,
                 bhhf, bhhr, Tc, K=8):
    T, B, _ = inputs[0].shape
    G = T // Tc
    n = len(inputs)

    in_specs = []
    for a in inputs:
        in_specs.append(
            pl.BlockSpec((Tc, B, a.shape[2]), lambda i: (i, 0, 0)))
    for a in inputs:
        in_specs.append(
            pl.BlockSpec((Tc, B, a.shape[2]), lambda i, G=G: (G - 1 - i, 0, 0)))
    for w in list(Wf_list) + list(Wr_list) + [WhhTf, WhhTr]:
        in_specs.append(
            pl.BlockSpec(w.shape, lambda i: (0, 0)))
    for b in (bihf, bihr, bhhf, bhhr):
        in_specs.append(pl.BlockSpec(b.shape, lambda i: (0, 0)))

    out_specs = [
        pl.BlockSpec((Tc, B, H), lambda i: (i, 0, 0)),
        pl.BlockSpec((Tc, B, H), lambda i, G=G: (G - 1 - i, 0, 0)),
    ]
    out_shape = [jax.ShapeDtypeStruct((T, B, H), BF16)] * 2
    scratch = [
        pltpu.VMEM((Tc, B, G3), F32),
        pltpu.VMEM((Tc, B, G3), F32),
        pltpu.VMEM((B, H), F32),
        pltpu.VMEM((B, H), F32),
    ]

    f, r = pl.pallas_call(
        functools.partial(_layer_body, n, Tc, K),
        grid=(G,),
        in_specs=in_specs,
        out_specs=out_specs,
        out_shape=out_shape,
        scratch_shapes=scratch,
        compiler_params=pltpu.CompilerParams(
            dimension_semantics=("arbitrary",)),
    )(*inputs, *inputs, *Wf_list, *Wr_list, WhhTf, WhhTr,
      bihf, bihr, bhhf, bhhr)
    return f, r


def _final_body(Tc, f_ref, r_ref, Af, Ar, b, out_ref):
    B = f_ref.shape[1]
    y = (jnp.dot(f_ref[...].reshape(Tc * B, H), Af[...],
                 preferred_element_type=F32)
         + jnp.dot(r_ref[...].reshape(Tc * B, H), Ar[...],
                   preferred_element_type=F32)
         + b[...])
    out_ref[...] = y.reshape(Tc, B, NCLS)


def _final_linear(f, r, W_fnl, b_fnl, Tc):
    T, B, _ = f.shape
    G = T // Tc
    WT = W_fnl.T.astype(BF16)
    Af = WT[:H]
    Ar = WT[H:]
    b2 = b_fnl.reshape(1, NCLS)

    return pl.pallas_call(
        functools.partial(_final_body, Tc),
        grid=(G,),
        in_specs=[
            pl.BlockSpec((Tc, B, H), lambda i: (i, 0, 0)),
            pl.BlockSpec((Tc, B, H), lambda i: (i, 0, 0)),
            pl.BlockSpec(Af.shape, lambda i: (0, 0)),
            pl.BlockSpec(Ar.shape, lambda i: (0, 0)),
            pl.BlockSpec(b2.shape, lambda i: (0, 0)),
        ],
        out_specs=pl.BlockSpec((Tc, B, NCLS), lambda i: (i, 0, 0)),
        out_shape=jax.ShapeDtypeStruct((T, B, NCLS), F32),
        compiler_params=pltpu.CompilerParams(
            dimension_semantics=("arbitrary",)),
    )(f, r, Af, Ar, b2)


def kernel(x, Wih_f0, Whh_f0, bih_f0, bhh_f0, Wih_r0, Whh_r0, bih_r0, bhh_r0,
           Wih_f1, Whh_f1, bih_f1, bhh_f1, Wih_r1, Whh_r1, bih_r1, bhh_r1,
           Wih_f2, Whh_f2, bih_f2, bhh_f2, Wih_r2, Whh_r2, bih_r2, bhh_r2,
           W_fnl, b_fnl):
    Tc = 64
    y = jnp.transpose(x, (2, 3, 0, 1))[0].astype(BF16)

    params = [
        (Wih_f0, Whh_f0, bih_f0, bhh_f0, Wih_r0, Whh_r0, bih_r0, bhh_r0),
        (Wih_f1, Whh_f1, bih_f1, bhh_f1, Wih_r1, Whh_r1, bih_r1, bhh_r1),
        (Wih_f2, Whh_f2, bih_f2, bhh_f2, Wih_r2, Whh_r2, bih_r2, bhh_r2),
    ]

    inputs = [y]
    for l, (Wif, Whf, bif, bhf, Wir, Whr, bir, bhr) in enumerate(params):
        WifT = Wif.T.astype(BF16)
        WirT = Wir.T.astype(BF16)
        if l == 0:
            Wf_list = [WifT]
            Wr_list = [WirT]
        else:
            Wf_list = [WifT[:H], WifT[H:]]
            Wr_list = [WirT[:H], WirT[H:]]
        zH = jnp.zeros((H,), F32)
        gbias_f = (bif + jnp.concatenate([bhf[:2 * H], zH])).reshape(1, G3)
        gbias_r = (bir + jnp.concatenate([bhr[:2 * H], zH])).reshape(1, G3)
        f, r = _bigru_layer(
            inputs, Wf_list, Wr_list, Whf.T.astype(BF16), Whr.T.astype(BF16),
            gbias_f, gbias_r,
            bhf[2 * H:].reshape(1, H), bhr[2 * H:].reshape(1, H), Tc)
        inputs = [f, r]

    return _final_linear(inputs[0], inputs[1], W_fnl, b_fnl, 64)


# speedup vs baseline: 1.2215x; 1.2215x over previous
import functools

import jax
import jax.numpy as jnp
from jax.experimental import pallas as pl
from jax.experimental.pallas import tpu as pltpu

H = 256
G3 = 3 * H
NCLS = 80
F32 = jnp.float32
BF16 = jnp.bfloat16


def _gru_update(gi, mm, bhn, h):
    grz = gi[:, :2 * H] + mm[:, :2 * H]
    r = jax.nn.sigmoid(grz[:, :H])
    z = jax.nn.sigmoid(grz[:, H:])
    n = jnp.tanh(gi[:, 2 * H:] + r * (mm[:, 2 * H:] + bhn))
    return z * (h - n) + n


def _layer_body(n_in, Tc, K, *refs):
    k = 0
    fwd = refs[k:k + n_in]; k += n_in
    rev = refs[k:k + n_in]; k += n_in
    Wf = refs[k:k + n_in]; k += n_in
    Wr = refs[k:k + n_in]; k += n_in
    WhhTf, WhhTr, bihf, bihr, bhhf, bhhr = refs[k:k + 6]; k += 6
    out_f, out_r = refs[k:k + 2]; k += 2
    gif_sc, gir_sc, hf_sc, hr_sc = refs[k:k + 4]

    B = out_f.shape[1]

    @pl.when(pl.program_id(0) == 0)
    def _():
        hf_sc[...] = jnp.zeros_like(hf_sc)
        hr_sc[...] = jnp.zeros_like(hr_sc)

    bf = bihf[...]
    br = bihr[...]
    wfv = [w[...] for w in Wf]
    wrv = [w[...] for w in Wr]

    NB = Tc // K

    gif = bf
    for a, wv in zip(fwd, wfv):
        x2 = a[0:2 * K].reshape(2 * K * B, a.shape[2])
        gif = gif + jnp.dot(x2, wv, preferred_element_type=F32)
    gif_sc[0:2 * K] = gif.reshape(2 * K, B, G3)

    gir = br
    for a, wv in zip(rev, wrv):
        x2 = a[Tc - 2 * K:Tc].reshape(2 * K * B, a.shape[2])
        gir = gir + jnp.dot(x2, wv, preferred_element_type=F32)
    gir_sc[Tc - 2 * K:Tc] = gir.reshape(2 * K, B, G3)

    whf = WhhTf[...]
    whr = WhhTr[...]
    bhnf = bhhf[...]
    bhnr = bhhr[...]

    def block(b, carry):
        hf, hr = carry
        tb = jnp.minimum(b + 2, NB - 1) * K
        acc = bf
        for a, wv in zip(fwd, wfv):
            x2 = a[pl.ds(tb, K)].reshape(K * B, a.shape[2])
            acc = acc + jnp.dot(x2, wv, preferred_element_type=F32)
        gif_sc[pl.ds(tb, K)] = acc.reshape(K, B, G3)

        tbr = Tc - tb - K
        accr = br
        for a, wv in zip(rev, wrv):
            x2 = a[pl.ds(tbr, K)].reshape(K * B, a.shape[2])
            accr = accr + jnp.dot(x2, wv, preferred_element_type=F32)
        gir_sc[pl.ds(tbr, K)] = accr.reshape(K, B, G3)

        for j in range(K):
            s = b * K + j
            mmf = jnp.dot(hf.astype(BF16), whf, preferred_element_type=F32)
            mmr = jnp.dot(hr.astype(BF16), whr, preferred_element_type=F32)
            hf = _gru_update(gif_sc[s], mmf, bhnf, hf)
            hr = _gru_update(gir_sc[Tc - 1 - s], mmr, bhnr, hr)
            out_f[s] = hf.astype(BF16)
            out_r[Tc - 1 - s] = hr.astype(BF16)
        return hf, hr

    hf, hr = jax.lax.fori_loop(0, NB, block, (hf_sc[...], hr_sc[...]))
    hf_sc[...] = hf
    hr_sc[...] = hr


def _bigru_layer(inputs, Wf_list, Wr_list, WhhTf, WhhTr, bihf, bihr,
                 bhhf, bhhr, Tc, K=8):
    T, B, _ = inputs[0].shape
    G = T // Tc
    n = len(inputs)

    in_specs = []
    for a in inputs:
        in_specs.append(
            pl.BlockSpec((Tc, B, a.shape[2]), lambda i: (i, 0, 0)))
    for a in inputs:
        in_specs.append(
            pl.BlockSpec((Tc, B, a.shape[2]), lambda i, G=G: (G - 1 - i, 0, 0)))
    for w in list(Wf_list) + list(Wr_list) + [WhhTf, WhhTr]:
        in_specs.append(
            pl.BlockSpec(w.shape, lambda i: (0, 0)))
    for b in (bihf, bihr, bhhf, bhhr):
        in_specs.append(pl.BlockSpec(b.shape, lambda i: (0, 0)))

    out_specs = [
        pl.BlockSpec((Tc, B, H), lambda i: (i, 0, 0)),
        pl.BlockSpec((Tc, B, H), lambda i, G=G: (G - 1 - i, 0, 0)),
    ]
    out_shape = [jax.ShapeDtypeStruct((T, B, H), BF16)] * 2
    scratch = [
        pltpu.VMEM((Tc, B, G3), F32),
        pltpu.VMEM((Tc, B, G3), F32),
        pltpu.VMEM((B, H), F32),
        pltpu.VMEM((B, H), F32),
    ]

    f, r = pl.pallas_call(
        functools.partial(_layer_body, n, Tc, K),
        grid=(G,),
        in_specs=in_specs,
        out_specs=out_specs,
        out_shape=out_shape,
        scratch_shapes=scratch,
        compiler_params=pltpu.CompilerParams(
            dimension_semantics=("arbitrary",)),
    )(*inputs, *inputs, *Wf_list, *Wr_list, WhhTf, WhhTr,
      bihf, bihr, bhhf, bhhr)
    return f, r


def _final_body(Tc, f_ref, r_ref, Af, Ar, b, out_ref):
    B = f_ref.shape[1]
    y = (jnp.dot(f_ref[...].reshape(Tc * B, H), Af[...],
                 preferred_element_type=F32)
         + jnp.dot(r_ref[...].reshape(Tc * B, H), Ar[...],
                   preferred_element_type=F32)
         + b[...])
    out_ref[...] = y.reshape(Tc, B, NCLS)


def _final_linear(f, r, W_fnl, b_fnl, Tc):
    T, B, _ = f.shape
    G = T // Tc
    WT = W_fnl.T.astype(BF16)
    Af = WT[:H]
    Ar = WT[H:]
    b2 = b_fnl.reshape(1, NCLS)

    return pl.pallas_call(
        functools.partial(_final_body, Tc),
        grid=(G,),
        in_specs=[
            pl.BlockSpec((Tc, B, H), lambda i: (i, 0, 0)),
            pl.BlockSpec((Tc, B, H), lambda i: (i, 0, 0)),
            pl.BlockSpec(Af.shape, lambda i: (0, 0)),
            pl.BlockSpec(Ar.shape, lambda i: (0, 0)),
            pl.BlockSpec(b2.shape, lambda i: (0, 0)),
        ],
        out_specs=pl.BlockSpec((Tc, B, NCLS), lambda i: (i, 0, 0)),
        out_shape=jax.ShapeDtypeStruct((T, B, NCLS), F32),
        compiler_params=pltpu.CompilerParams(
            dimension_semantics=("arbitrary",)),
    )(f, r, Af, Ar, b2)


def kernel(x, Wih_f0, Whh_f0, bih_f0, bhh_f0, Wih_r0, Whh_r0, bih_r0, bhh_r0,
           Wih_f1, Whh_f1, bih_f1, bhh_f1, Wih_r1, Whh_r1, bih_r1, bhh_r1,
           Wih_f2, Whh_f2, bih_f2, bhh_f2, Wih_r2, Whh_r2, bih_r2, bhh_r2,
           W_fnl, b_fnl):
    Tc = 64
    y = jnp.transpose(x, (2, 3, 0, 1))[0].astype(BF16)

    params = [
        (Wih_f0, Whh_f0, bih_f0, bhh_f0, Wih_r0, Whh_r0, bih_r0, bhh_r0),
        (Wih_f1, Whh_f1, bih_f1, bhh_f1, Wih_r1, Whh_r1, bih_r1, bhh_r1),
        (Wih_f2, Whh_f2, bih_f2, bhh_f2, Wih_r2, Whh_r2, bih_r2, bhh_r2),
    ]

    inputs = [y]
    for l, (Wif, Whf, bif, bhf, Wir, Whr, bir, bhr) in enumerate(params):
        WifT = Wif.T.astype(BF16)
        WirT = Wir.T.astype(BF16)
        if l == 0:
            Wf_list = [WifT]
            Wr_list = [WirT]
        else:
            Wf_list = [WifT[:H], WifT[H:]]
            Wr_list = [WirT[:H], WirT[H:]]
        zH = jnp.zeros((H,), F32)
        gbias_f = (bif + jnp.concatenate([bhf[:2 * H], zH])).reshape(1, G3)
        gbias_r = (bir + jnp.concatenate([bhr[:2 * H], zH])).reshape(1, G3)
        f, r = _bigru_layer(
            inputs, Wf_list, Wr_list, Whf.T.astype(BF16), Whr.T.astype(BF16),
            gbias_f, gbias_r,
            bhf[2 * H:].reshape(1, H), bhr[2 * H:].reshape(1, H), Tc)
        inputs = [f, r]

    return _final_linear(inputs[0], inputs[1], W_fnl, b_fnl, 64)


# speedup vs baseline: 1.2671x; 1.0373x over previous
import functools

import jax
import jax.numpy as jnp
from jax.experimental import pallas as pl
from jax.experimental.pallas import tpu as pltpu

H = 256
G3 = 3 * H
NCLS = 80
F32 = jnp.float32
BF16 = jnp.bfloat16


def _gru_update(gi, mm, bhn, h):
    grz = gi[:, :2 * H] + mm[:, :2 * H]
    r = jax.nn.sigmoid(grz[:, :H])
    z = jax.nn.sigmoid(grz[:, H:])
    n = jnp.tanh(gi[:, 2 * H:] + r * (mm[:, 2 * H:] + bhn))
    return z * (h - n) + n


def _layer_body(n_in, Tc, K, *refs):
    k = 0
    fwd = refs[k:k + n_in]; k += n_in
    rev = refs[k:k + n_in]; k += n_in
    Wf = refs[k:k + n_in]; k += n_in
    Wr = refs[k:k + n_in]; k += n_in
    WhhTf, WhhTr, bihf, bihr, bhhf, bhhr = refs[k:k + 6]; k += 6
    out_f, out_r = refs[k:k + 2]; k += 2
    gif_sc, gir_sc, hf_sc, hr_sc = refs[k:k + 4]

    B = out_f.shape[1]

    @pl.when(pl.program_id(0) == 0)
    def _():
        hf_sc[...] = jnp.zeros_like(hf_sc)
        hr_sc[...] = jnp.zeros_like(hr_sc)

    bf = bihf[...]
    br = bihr[...]
    wfv = [w[...] for w in Wf]
    wrv = [w[...] for w in Wr]

    NB = Tc // K

    gif = bf
    for a, wv in zip(fwd, wfv):
        x2 = a[0:2 * K].reshape(2 * K * B, a.shape[2])
        gif = gif + jnp.dot(x2, wv, preferred_element_type=F32)
    gif_sc[0:2 * K] = gif.reshape(2 * K, B, G3)

    gir = br
    for a, wv in zip(rev, wrv):
        x2 = a[Tc - 2 * K:Tc].reshape(2 * K * B, a.shape[2])
        gir = gir + jnp.dot(x2, wv, preferred_element_type=F32)
    girv = gir.reshape(2 * K, B, G3)
    gir_sc[0:K] = girv[K:2 * K]
    gir_sc[K:2 * K] = girv[0:K]

    whf = WhhTf[...]
    whr = WhhTr[...]
    bhnf = bhhf[...]
    bhnr = bhhr[...]

    def quad(ob, carry):
        hf, hr = carry
        for p in range(4):
            b = ob * 4 + p
            wslot = (p + 2) % 4
            tb = jnp.minimum(b + 2, NB - 1) * K
            acc = bf
            for a, wv in zip(fwd, wfv):
                x2 = a[pl.ds(tb, K)].reshape(K * B, a.shape[2])
                acc = acc + jnp.dot(x2, wv, preferred_element_type=F32)
            gif_sc[wslot * K:(wslot + 1) * K] = acc.reshape(K, B, G3)

            tbr = Tc - tb - K
            accr = br
            for a, wv in zip(rev, wrv):
                x2 = a[pl.ds(tbr, K)].reshape(K * B, a.shape[2])
                accr = accr + jnp.dot(x2, wv, preferred_element_type=F32)
            gir_sc[wslot * K:(wslot + 1) * K] = accr.reshape(K, B, G3)

            for j in range(K):
                s = b * K + j
                mmf = jnp.dot(hf.astype(BF16), whf,
                              preferred_element_type=F32)
                mmr = jnp.dot(hr.astype(BF16), whr,
                              preferred_element_type=F32)
                hf = _gru_update(gif_sc[p * K + j], mmf, bhnf, hf)
                hr = _gru_update(gir_sc[p * K + (K - 1 - j)], mmr, bhnr, hr)
                out_f[s] = hf.astype(BF16)
                out_r[Tc - 1 - s] = hr.astype(BF16)
        return hf, hr

    hf, hr = jax.lax.fori_loop(0, NB // 4, quad, (hf_sc[...], hr_sc[...]))
    hf_sc[...] = hf
    hr_sc[...] = hr


def _bigru_layer(inputs, Wf_list, Wr_list, WhhTf, WhhTr, bihf, bihr,
                 bhhf, bhhr, Tc, K=8):
    T, B, _ = inputs[0].shape
    G = T // Tc
    n = len(inputs)

    in_specs = []
    for a in inputs:
        in_specs.append(
            pl.BlockSpec((Tc, B, a.shape[2]), lambda i: (i, 0, 0)))
    for a in inputs:
        in_specs.append(
            pl.BlockSpec((Tc, B, a.shape[2]), lambda i, G=G: (G - 1 - i, 0, 0)))
    for w in list(Wf_list) + list(Wr_list) + [WhhTf, WhhTr]:
        in_specs.append(
            pl.BlockSpec(w.shape, lambda i: (0, 0)))
    for b in (bihf, bihr, bhhf, bhhr):
        in_specs.append(pl.BlockSpec(b.shape, lambda i: (0, 0)))

    out_specs = [
        pl.BlockSpec((Tc, B, H), lambda i: (i, 0, 0)),
        pl.BlockSpec((Tc, B, H), lambda i, G=G: (G - 1 - i, 0, 0)),
    ]
    out_shape = [jax.ShapeDtypeStruct((T, B, H), BF16)] * 2
    scratch = [
        pltpu.VMEM((4 * K, B, G3), F32),
        pltpu.VMEM((4 * K, B, G3), F32),
        pltpu.VMEM((B, H), F32),
        pltpu.VMEM((B, H), F32),
    ]

    f, r = pl.pallas_call(
        functools.partial(_layer_body, n, Tc, K),
        grid=(G,),
        in_specs=in_specs,
        out_specs=out_specs,
        out_shape=out_shape,
        scratch_shapes=scratch,
        compiler_params=pltpu.CompilerParams(
            dimension_semantics=("arbitrary",)),
    )(*inputs, *inputs, *Wf_list, *Wr_list, WhhTf, WhhTr,
      bihf, bihr, bhhf, bhhr)
    return f, r


def _final_body(Tc, f_ref, r_ref, Af, Ar, b, out_ref):
    B = f_ref.shape[1]
    y = (jnp.dot(f_ref[...].reshape(Tc * B, H), Af[...],
                 preferred_element_type=F32)
         + jnp.dot(r_ref[...].reshape(Tc * B, H), Ar[...],
                   preferred_element_type=F32)
         + b[...])
    out_ref[...] = y.reshape(Tc, B, NCLS)


def _final_linear(f, r, W_fnl, b_fnl, Tc):
    T, B, _ = f.shape
    G = T // Tc
    WT = W_fnl.T.astype(BF16)
    Af = WT[:H]
    Ar = WT[H:]
    b2 = b_fnl.reshape(1, NCLS)

    return pl.pallas_call(
        functools.partial(_final_body, Tc),
        grid=(G,),
        in_specs=[
            pl.BlockSpec((Tc, B, H), lambda i: (i, 0, 0)),
            pl.BlockSpec((Tc, B, H), lambda i: (i, 0, 0)),
            pl.BlockSpec(Af.shape, lambda i: (0, 0)),
            pl.BlockSpec(Ar.shape, lambda i: (0, 0)),
            pl.BlockSpec(b2.shape, lambda i: (0, 0)),
        ],
        out_specs=pl.BlockSpec((Tc, B, NCLS), lambda i: (i, 0, 0)),
        out_shape=jax.ShapeDtypeStruct((T, B, NCLS), F32),
        compiler_params=pltpu.CompilerParams(
            dimension_semantics=("arbitrary",)),
    )(f, r, Af, Ar, b2)


def kernel(x, Wih_f0, Whh_f0, bih_f0, bhh_f0, Wih_r0, Whh_r0, bih_r0, bhh_r0,
           Wih_f1, Whh_f1, bih_f1, bhh_f1, Wih_r1, Whh_r1, bih_r1, bhh_r1,
           Wih_f2, Whh_f2, bih_f2, bhh_f2, Wih_r2, Whh_r2, bih_r2, bhh_r2,
           W_fnl, b_fnl):
    Tc = 64
    y = jnp.transpose(x, (2, 3, 0, 1))[0].astype(BF16)

    params = [
        (Wih_f0, Whh_f0, bih_f0, bhh_f0, Wih_r0, Whh_r0, bih_r0, bhh_r0),
        (Wih_f1, Whh_f1, bih_f1, bhh_f1, Wih_r1, Whh_r1, bih_r1, bhh_r1),
        (Wih_f2, Whh_f2, bih_f2, bhh_f2, Wih_r2, Whh_r2, bih_r2, bhh_r2),
    ]

    inputs = [y]
    for l, (Wif, Whf, bif, bhf, Wir, Whr, bir, bhr) in enumerate(params):
        WifT = Wif.T.astype(BF16)
        WirT = Wir.T.astype(BF16)
        if l == 0:
            Wf_list = [WifT]
            Wr_list = [WirT]
        else:
            Wf_list = [WifT[:H], WifT[H:]]
            Wr_list = [WirT[:H], WirT[H:]]
        zH = jnp.zeros((H,), F32)
        gbias_f = (bif + jnp.concatenate([bhf[:2 * H], zH])).reshape(1, G3)
        gbias_r = (bir + jnp.concatenate([bhr[:2 * H], zH])).reshape(1, G3)
        f, r = _bigru_layer(
            inputs, Wf_list, Wr_list, Whf.T.astype(BF16), Whr.T.astype(BF16),
            gbias_f, gbias_r,
            bhf[2 * H:].reshape(1, H), bhr[2 * H:].reshape(1, H), Tc)
        inputs = [f, r]

    return _final_linear(inputs[0], inputs[1], W_fnl, b_fnl, 64)


# speedup vs baseline: 1.2940x; 1.0212x over previous
import functools

import jax
import jax.numpy as jnp
from jax.experimental import pallas as pl
from jax.experimental.pallas import tpu as pltpu

H = 256
G3 = 3 * H
NCLS = 80
F32 = jnp.float32
BF16 = jnp.bfloat16


def _gru_update(gi, mm, bhn, h):
    grz = gi[:, :2 * H] + mm[:, :2 * H]
    r = jax.nn.sigmoid(grz[:, :H])
    z = jax.nn.sigmoid(grz[:, H:])
    n = jnp.tanh(gi[:, 2 * H:] + r * (mm[:, 2 * H:] + bhn))
    return z * (h - n) + n


def _layer_body(n_in, Tc, raw, *refs):
    k = 0
    fwd = refs[k:k + n_in]; k += n_in
    rev = refs[k:k + n_in]; k += n_in
    Wf = refs[k:k + n_in]; k += n_in
    Wr = refs[k:k + n_in]; k += n_in
    WhhTf, WhhTr, bihf, bihr, bhhf, bhhr = refs[k:k + 6]; k += 6
    out_f, out_r = refs[k:k + 2]; k += 2
    gif_sc, gir_sc, hf_sc, hr_sc = refs[k:k + 4]; k += 4

    B = out_f.shape[1]

    @pl.when(pl.program_id(0) == 0)
    def _():
        hf_sc[...] = jnp.zeros_like(hf_sc)
        hr_sc[...] = jnp.zeros_like(hr_sc)

    if raw:
        xtf_sc, xtr_sc = refs[k:k + 2]
        xtf_sc[...] = jnp.transpose(fwd[0][...].astype(BF16), (2, 0, 1))
        xtr_sc[...] = jnp.transpose(rev[0][...].astype(BF16), (2, 0, 1))
        fwd = [xtf_sc]
        rev = [xtr_sc]

    gif = bihf[...]
    for a, w in zip(fwd, Wf):
        x2 = a[...].reshape(Tc * B, a.shape[2])
        gif = gif + jnp.dot(x2, w[...], preferred_element_type=F32)
    gif_sc[...] = gif.reshape(Tc, B, G3)

    gir = bihr[...]
    for a, w in zip(rev, Wr):
        x2 = a[...].reshape(Tc * B, a.shape[2])
        gir = gir + jnp.dot(x2, w[...], preferred_element_type=F32)
    gir_sc[...] = gir.reshape(Tc, B, G3)

    whf = WhhTf[...]
    whr = WhhTr[...]
    bhnf = bhhf[...]
    bhnr = bhhr[...]

    def step(s, carry):
        hf, hr = carry
        mmf = jnp.dot(hf.astype(BF16), whf, preferred_element_type=F32)
        mmr = jnp.dot(hr.astype(BF16), whr, preferred_element_type=F32)
        hf = _gru_update(gif_sc[s], mmf, bhnf, hf)
        hr = _gru_update(gir_sc[Tc - 1 - s], mmr, bhnr, hr)
        out_f[s] = hf.astype(BF16)
        out_r[Tc - 1 - s] = hr.astype(BF16)
        return hf, hr

    hf, hr = jax.lax.fori_loop(0, Tc, step, (hf_sc[...], hr_sc[...]),
                               unroll=8)
    hf_sc[...] = hf
    hr_sc[...] = hr


def _bigru_layer(inputs, Wf_list, Wr_list, WhhTf, WhhTr, bihf, bihr,
                 bhhf, bhhr, Tc, raw=False):
    if raw:
        B, C, T = inputs[0].shape
    else:
        T, B, _ = inputs[0].shape
    G = T // Tc
    n = len(inputs)

    in_specs = []
    if raw:
        in_specs.append(
            pl.BlockSpec((B, C, Tc), lambda i: (0, 0, i)))
        in_specs.append(
            pl.BlockSpec((B, C, Tc), lambda i, G=G: (0, 0, G - 1 - i)))
    else:
        for a in inputs:
            in_specs.append(
                pl.BlockSpec((Tc, B, a.shape[2]), lambda i: (i, 0, 0)))
        for a in inputs:
            in_specs.append(
                pl.BlockSpec((Tc, B, a.shape[2]),
                             lambda i, G=G: (G - 1 - i, 0, 0)))
    for w in list(Wf_list) + list(Wr_list) + [WhhTf, WhhTr]:
        in_specs.append(
            pl.BlockSpec(w.shape, lambda i: (0, 0)))
    for b in (bihf, bihr, bhhf, bhhr):
        in_specs.append(pl.BlockSpec(b.shape, lambda i: (0, 0)))

    out_specs = [
        pl.BlockSpec((Tc, B, H), lambda i: (i, 0, 0)),
        pl.BlockSpec((Tc, B, H), lambda i, G=G: (G - 1 - i, 0, 0)),
    ]
    out_shape = [jax.ShapeDtypeStruct((T, B, H), BF16)] * 2
    scratch = [
        pltpu.VMEM((Tc, B, G3), F32),
        pltpu.VMEM((Tc, B, G3), F32),
        pltpu.VMEM((B, H), F32),
        pltpu.VMEM((B, H), F32),
    ]
    if raw:
        scratch += [
            pltpu.VMEM((Tc, B, C), BF16),
            pltpu.VMEM((Tc, B, C), BF16),
        ]

    f, r = pl.pallas_call(
        functools.partial(_layer_body, n, Tc, raw),
        grid=(G,),
        in_specs=in_specs,
        out_specs=out_specs,
        out_shape=out_shape,
        scratch_shapes=scratch,
        compiler_params=pltpu.CompilerParams(
            dimension_semantics=("arbitrary",)),
    )(*inputs, *inputs, *Wf_list, *Wr_list, WhhTf, WhhTr,
      bihf, bihr, bhhf, bhhr)
    return f, r


def _transpose_body(in_ref, out_ref):
    out_ref[...] = jnp.transpose(in_ref[...].astype(BF16), (2, 0, 1))


def _transpose_cast(x3, Tt):
    B, C, T = x3.shape
    return pl.pallas_call(
        _transpose_body,
        grid=(T // Tt,),
        in_specs=[pl.BlockSpec((B, C, Tt), lambda i: (0, 0, i))],
        out_specs=pl.BlockSpec((Tt, B, C), lambda i: (i, 0, 0)),
        out_shape=jax.ShapeDtypeStruct((T, B, C), BF16),
    )(x3)


def _final_body(Tc, f_ref, r_ref, Af, Ar, b, out_ref):
    B = f_ref.shape[1]
    y = (jnp.dot(f_ref[...].reshape(Tc * B, H), Af[...],
                 preferred_element_type=F32)
         + jnp.dot(r_ref[...].reshape(Tc * B, H), Ar[...],
                   preferred_element_type=F32)
         + b[...])
    out_ref[...] = y.reshape(Tc, B, NCLS)


def _final_linear(f, r, W_fnl, b_fnl, Tc):
    T, B, _ = f.shape
    G = T // Tc
    WT = W_fnl.T.astype(BF16)
    Af = WT[:H]
    Ar = WT[H:]
    b2 = b_fnl.reshape(1, NCLS)

    return pl.pallas_call(
        functools.partial(_final_body, Tc),
        grid=(G,),
        in_specs=[
            pl.BlockSpec((Tc, B, H), lambda i: (i, 0, 0)),
            pl.BlockSpec((Tc, B, H), lambda i: (i, 0, 0)),
            pl.BlockSpec(Af.shape, lambda i: (0, 0)),
            pl.BlockSpec(Ar.shape, lambda i: (0, 0)),
            pl.BlockSpec(b2.shape, lambda i: (0, 0)),
        ],
        out_specs=pl.BlockSpec((Tc, B, NCLS), lambda i: (i, 0, 0)),
        out_shape=jax.ShapeDtypeStruct((T, B, NCLS), F32),
        compiler_params=pltpu.CompilerParams(
            dimension_semantics=("arbitrary",)),
    )(f, r, Af, Ar, b2)


def kernel(x, Wih_f0, Whh_f0, bih_f0, bhh_f0, Wih_r0, Whh_r0, bih_r0, bhh_r0,
           Wih_f1, Whh_f1, bih_f1, bhh_f1, Wih_r1, Whh_r1, bih_r1, bhh_r1,
           Wih_f2, Whh_f2, bih_f2, bhh_f2, Wih_r2, Whh_r2, bih_r2, bhh_r2,
           W_fnl, b_fnl):
    Tc = 64
    x3 = x.reshape(x.shape[0], x.shape[1], x.shape[3])
    y = _transpose_cast(x3, 128)

    params = [
        (Wih_f0, Whh_f0, bih_f0, bhh_f0, Wih_r0, Whh_r0, bih_r0, bhh_r0),
        (Wih_f1, Whh_f1, bih_f1, bhh_f1, Wih_r1, Whh_r1, bih_r1, bhh_r1),
        (Wih_f2, Whh_f2, bih_f2, bhh_f2, Wih_r2, Whh_r2, bih_r2, bhh_r2),
    ]

    inputs = [y]
    for l, (Wif, Whf, bif, bhf, Wir, Whr, bir, bhr) in enumerate(params):
        WifT = Wif.T.astype(BF16)
        WirT = Wir.T.astype(BF16)
        if l == 0:
            Wf_list = [WifT]
            Wr_list = [WirT]
        else:
            Wf_list = [WifT[:H], WifT[H:]]
            Wr_list = [WirT[:H], WirT[H:]]
        zH = jnp.zeros((H,), F32)
        gbias_f = (bif + jnp.concatenate([bhf[:2 * H], zH])).reshape(1, G3)
        gbias_r = (bir + jnp.concatenate([bhr[:2 * H], zH])).reshape(1, G3)
        f, r = _bigru_layer(
            inputs, Wf_list, Wr_list, Whf.T.astype(BF16), Whr.T.astype(BF16),
            gbias_f, gbias_r,
            bhf[2 * H:].reshape(1, H), bhr[2 * H:].reshape(1, H), Tc)
        inputs = [f, r]

    return _final_linear(inputs[0], inputs[1], W_fnl, b_fnl, 64)


# speedup vs baseline: 1.3217x; 1.0214x over previous
import functools

import jax
import jax.numpy as jnp
from jax.experimental import pallas as pl
from jax.experimental.pallas import tpu as pltpu

H = 256
G3 = 3 * H
NCLS = 80
F32 = jnp.float32
BF16 = jnp.bfloat16


def _gru_update(gi, mm, bhn, h):
    grz = gi[:, :2 * H] + mm[:, :2 * H]
    r = jax.nn.sigmoid(grz[:, :H])
    z = jax.nn.sigmoid(grz[:, H:])
    n = jnp.tanh(gi[:, 2 * H:] + r * (mm[:, 2 * H:] + bhn))
    return z * (h - n) + n


def _layer_body(n_in, Tc, *refs):
    k = 0
    fwd = refs[k:k + n_in]; k += n_in
    rev = refs[k:k + n_in]; k += n_in
    Wf = refs[k:k + n_in]; k += n_in
    Wr = refs[k:k + n_in]; k += n_in
    WhhTf, WhhTr, bihf, bihr, bhhf, bhhr = refs[k:k + 6]; k += 6
    out_f, out_r = refs[k:k + 2]; k += 2
    gif_sc, gir_sc, hf_sc, hr_sc = refs[k:k + 4]

    B = out_f.shape[1]

    @pl.when(pl.program_id(0) == 0)
    def _():
        hf_sc[...] = jnp.zeros_like(hf_sc)
        hr_sc[...] = jnp.zeros_like(hr_sc)

    gif = bihf[...]
    for a, w in zip(fwd, Wf):
        x2 = a[...].reshape(Tc * B, a.shape[2])
        gif = gif + jnp.dot(x2, w[...], preferred_element_type=F32)
    gif_sc[...] = gif.reshape(Tc, B, G3)

    gir = bihr[...]
    for a, w in zip(rev, Wr):
        x2 = a[...].reshape(Tc * B, a.shape[2])
        gir = gir + jnp.dot(x2, w[...], preferred_element_type=F32)
    gir_sc[...] = gir.reshape(Tc, B, G3)

    whf = WhhTf[...]
    whr = WhhTr[...]
    bhnf = bhhf[...]
    bhnr = bhhr[...]

    def step(s, carry):
        hf, hr = carry
        mmf = jnp.dot(hf.astype(BF16), whf, preferred_element_type=F32)
        mmr = jnp.dot(hr.astype(BF16), whr, preferred_element_type=F32)
        hf = _gru_update(gif_sc[s], mmf, bhnf, hf)
        hr = _gru_update(gir_sc[Tc - 1 - s], mmr, bhnr, hr)
        out_f[s] = hf.astype(BF16)
        out_r[Tc - 1 - s] = hr.astype(BF16)
        return hf, hr

    hf, hr = jax.lax.fori_loop(0, Tc, step, (hf_sc[...], hr_sc[...]),
                               unroll=16)
    hf_sc[...] = hf
    hr_sc[...] = hr


def _bigru_layer(inputs, Wf_list, Wr_list, WhhTf, WhhTr, bihf, bihr,
                 bhhf, bhhr, Tc):
    T, B, _ = inputs[0].shape
    G = T // Tc
    n = len(inputs)

    in_specs = []
    for a in inputs:
        in_specs.append(
            pl.BlockSpec((Tc, B, a.shape[2]), lambda i: (i, 0, 0)))
    for a in inputs:
        in_specs.append(
            pl.BlockSpec((Tc, B, a.shape[2]), lambda i, G=G: (G - 1 - i, 0, 0)))
    for w in list(Wf_list) + list(Wr_list) + [WhhTf, WhhTr]:
        in_specs.append(
            pl.BlockSpec(w.shape, lambda i: (0, 0)))
    for b in (bihf, bihr, bhhf, bhhr):
        in_specs.append(pl.BlockSpec(b.shape, lambda i: (0, 0)))

    out_specs = [
        pl.BlockSpec((Tc, B, H), lambda i: (i, 0, 0)),
        pl.BlockSpec((Tc, B, H), lambda i, G=G: (G - 1 - i, 0, 0)),
    ]
    out_shape = [jax.ShapeDtypeStruct((T, B, H), BF16)] * 2
    scratch = [
        pltpu.VMEM((Tc, B, G3), F32),
        pltpu.VMEM((Tc, B, G3), F32),
        pltpu.VMEM((B, H), F32),
        pltpu.VMEM((B, H), F32),
    ]

    f, r = pl.pallas_call(
        functools.partial(_layer_body, n, Tc),
        grid=(G,),
        in_specs=in_specs,
        out_specs=out_specs,
        out_shape=out_shape,
        scratch_shapes=scratch,
        compiler_params=pltpu.CompilerParams(
            dimension_semantics=("arbitrary",)),
    )(*inputs, *inputs, *Wf_list, *Wr_list, WhhTf, WhhTr,
      bihf, bihr, bhhf, bhhr)
    return f, r


def _final_body(Tc, f_ref, r_ref, Af, Ar, b, out_ref):
    B = f_ref.shape[1]
    y = (jnp.dot(f_ref[...].reshape(Tc * B, H), Af[...],
                 preferred_element_type=F32)
         + jnp.dot(r_ref[...].reshape(Tc * B, H), Ar[...],
                   preferred_element_type=F32)
         + b[...])
    out_ref[...] = y.reshape(Tc, B, NCLS)


def _final_linear(f, r, W_fnl, b_fnl, Tc):
    T, B, _ = f.shape
    G = T // Tc
    WT = W_fnl.T.astype(BF16)
    Af = WT[:H]
    Ar = WT[H:]
    b2 = b_fnl.reshape(1, NCLS)

    return pl.pallas_call(
        functools.partial(_final_body, Tc),
        grid=(G,),
        in_specs=[
            pl.BlockSpec((Tc, B, H), lambda i: (i, 0, 0)),
            pl.BlockSpec((Tc, B, H), lambda i: (i, 0, 0)),
            pl.BlockSpec(Af.shape, lambda i: (0, 0)),
            pl.BlockSpec(Ar.shape, lambda i: (0, 0)),
            pl.BlockSpec(b2.shape, lambda i: (0, 0)),
        ],
        out_specs=pl.BlockSpec((Tc, B, NCLS), lambda i: (i, 0, 0)),
        out_shape=jax.ShapeDtypeStruct((T, B, NCLS), F32),
        compiler_params=pltpu.CompilerParams(
            dimension_semantics=("arbitrary",)),
    )(f, r, Af, Ar, b2)


def kernel(x, Wih_f0, Whh_f0, bih_f0, bhh_f0, Wih_r0, Whh_r0, bih_r0, bhh_r0,
           Wih_f1, Whh_f1, bih_f1, bhh_f1, Wih_r1, Whh_r1, bih_r1, bhh_r1,
           Wih_f2, Whh_f2, bih_f2, bhh_f2, Wih_r2, Whh_r2, bih_r2, bhh_r2,
           W_fnl, b_fnl):
    Tc = 64
    y = jnp.transpose(x, (2, 3, 0, 1))[0].astype(BF16)

    params = [
        (Wih_f0, Whh_f0, bih_f0, bhh_f0, Wih_r0, Whh_r0, bih_r0, bhh_r0),
        (Wih_f1, Whh_f1, bih_f1, bhh_f1, Wih_r1, Whh_r1, bih_r1, bhh_r1),
        (Wih_f2, Whh_f2, bih_f2, bhh_f2, Wih_r2, Whh_r2, bih_r2, bhh_r2),
    ]

    inputs = [y]
    for l, (Wif, Whf, bif, bhf, Wir, Whr, bir, bhr) in enumerate(params):
        WifT = Wif.T.astype(BF16)
        WirT = Wir.T.astype(BF16)
        if l == 0:
            Wf_list = [WifT]
            Wr_list = [WirT]
        else:
            Wf_list = [WifT[:H], WifT[H:]]
            Wr_list = [WirT[:H], WirT[H:]]
        zH = jnp.zeros((H,), F32)
        gbias_f = (bif + jnp.concatenate([bhf[:2 * H], zH])).reshape(1, G3)
        gbias_r = (bir + jnp.concatenate([bhr[:2 * H], zH])).reshape(1, G3)
        f, r = _bigru_layer(
            inputs, Wf_list, Wr_list, Whf.T.astype(BF16), Whr.T.astype(BF16),
            gbias_f, gbias_r,
            bhf[2 * H:].reshape(1, H), bhr[2 * H:].reshape(1, H), Tc)
        inputs = [f, r]

    return _final_linear(inputs[0], inputs[1], W_fnl, b_fnl, 64)
